# baseline (speedup 1.0000x reference)
import functools

import numpy as np
import jax
import jax.numpy as jnp
from jax import lax
from jax.experimental import pallas as pl
from jax.experimental.pallas import tpu as pltpu
from jax.experimental.pallas import tpu_sc as plsc

_NL = 16
_T = 2 ** 19
_B = 131072
_P1 = np.int32(np.uint32(2654435761))
_P2 = np.int32(np.uint32(805459861))
_RES = [int(np.floor(16 * 1.4472692012786865 ** l)) for l in range(_NL)]
_DENSE = [(r + 1) ** 3 <= _T for r in _RES]

_NC, _NS = 2, 16
_NW = _NC * _NS
_PW = _B // _NW
_NCH = _PW // 16
_CT = 4096
_CPL = _T // _CT
_PPW = (_NL * _CPL) // _NW


def _mesh():
    return plsc.VectorSubcoreMesh(core_axis_name="c", subcore_axis_name="s")


def _relayout_body(tbl, out, buf):
    wid = lax.axis_index("s") * _NC + lax.axis_index("c")

    def body(i, carry):
        piece = wid * _PPW + i
        l = piece // _CPL
        c = piece - l * _CPL
        r0 = c * _CT
        for h in range(4):
            pltpu.sync_copy(tbl.at[h, l, pl.ds(r0, _CT), :],
                            buf.at[:, pl.ds(2 * h, 2)])
        pltpu.sync_copy(buf, out.at[pl.ds(l * _T + r0, _CT), :])
        return carry

    lax.fori_loop(0, _PPW, body, 0)


def _lookup_body(Tt, xT, cT, out, xb, cb, idxb, wcb, rows, obuf, gsem):
    wid = lax.axis_index("s") * _NC + lax.axis_index("c")
    base = wid * _PW
    pltpu.sync_copy(xT.at[:, pl.ds(base, _PW)], xb)
    pltpu.sync_copy(cT.at[:, pl.ds(base, _PW)], cb)
    lane = lax.iota(jnp.int32, 16)
    cols = [jnp.full((16,), v, jnp.int32) for v in range(8)]

    def chunk(ch, carry):
        off = ch * 16
        xs = [xb[d, pl.ds(off, 16)] for d in range(3)]
        copies = []
        for l in range(_NL):
            res = _RES[l]
            pos = [xs[d] * jnp.float32(res) for d in range(3)]
            ip = [p.astype(jnp.int32) for p in pos]
            w1 = [pos[d] - ip[d].astype(jnp.float32) for d in range(3)]
            w0 = [jnp.float32(1.0) - w1[d] for d in range(3)]
            c0 = ip
            c1 = [ip[d] + 1 for d in range(3)]
            if _DENSE[l]:
                s = np.int32(res + 1)
                sc2 = [c0[2] * s, c1[2] * s]
                u = [[(c1[1] if b1 else c0[1]) + sc2[b2] for b2 in range(2)]
                     for b1 in range(2)]

                def cidx(b0, b1, b2, s=s, u=u, c0=c0, c1=c1):
                    return (c1[0] if b0 else c0[0]) + s * u[b1][b2]
            else:
                h1 = [c0[1] * _P1, c1[1] * _P1]
                h2 = [c0[2] * _P2, c1[2] * _P2]

                def cidx(b0, b1, b2, h1=h1, h2=h2, c0=c0, c1=c1):
                    h = (c1[0] if b0 else c0[0]) ^ h1[b1] ^ h2[b2]
                    return h & np.int32(_T - 1)

            lbase = np.int32(l * _T)
            p = [[(w1[1] if b1 else w0[1]) * (w1[2] if b2 else w0[2])
                  for b2 in range(2)] for b1 in range(2)]
            for corner in range(8):
                b0, b1, b2 = corner & 1, (corner >> 1) & 1, (corner >> 2) & 1
                o = l * 128 + corner * 16
                idxb[pl.ds(o, 16)] = cidx(b0, b1, b2) + lbase
                wcb[pl.ds(o, 16)] = (w1[0] if b0 else w0[0]) * p[b1][b2]
            copies.append(pltpu.async_copy(
                Tt.at[idxb.at[pl.ds(l * 128, 128)]],
                rows.at[pl.ds(l * 128, 128), :], gsem))
        for cp in copies:
            cp.wait()
        code = [cb[h, pl.ds(off, 16)] for h in range(4)]
        for l in range(_NL):
            rowl = lane + np.int32(l * 128)
            acc = [None, None]
            for corner in range(8):
                rc = rowl + np.int32(corner * 16)
                wc = wcb[pl.ds(l * 128 + corner * 16, 16)]
                for f in range(2):
                    t = code[0] * plsc.load_gather(rows, [rc, cols[f]])
                    for h in range(1, 4):
                        t = t + code[h] * plsc.load_gather(rows, [rc, cols[h * 2 + f]])
                    wt = wc * t
                    acc[f] = wt if acc[f] is None else acc[f] + wt
            for f in range(2):
                plsc.store_scatter(obuf, [lane, jnp.full((16,), 2 * l + f, jnp.int32)],
                                   acc[f])
        pltpu.sync_copy(obuf, out.at[pl.ds(base + off, 16), :])
        return carry

    lax.fori_loop(0, _NCH, chunk, 0)


def kernel(in_tensor, conditioning_code, tables):
    xT = in_tensor.T
    cT = conditioning_code.T
    relayout = pl.kernel(
        _relayout_body, mesh=_mesh(),
        out_type=jax.ShapeDtypeStruct((_NL * _T, 8), jnp.float32),
        scratch_types=[pltpu.VMEM((_CT, 8), jnp.float32)],
        compiler_params=pltpu.CompilerParams(use_tc_tiling_on_sc=False),
    )
    Tt = relayout(tables)
    lookup = pl.kernel(
        _lookup_body, mesh=_mesh(),
        out_type=jax.ShapeDtypeStruct((_B, 32), jnp.float32),
        scratch_types=[
            pltpu.VMEM((3, _PW), jnp.float32),
            pltpu.VMEM((4, _PW), jnp.float32),
            pltpu.VMEM((_NL * 128,), jnp.int32),
            pltpu.VMEM((_NL * 128,), jnp.float32),
            pltpu.VMEM((_NL * 128, 8), jnp.float32),
            pltpu.VMEM((16, 32), jnp.float32),
            pltpu.SemaphoreType.DMA,
        ],
        compiler_params=pltpu.CompilerParams(use_tc_tiling_on_sc=False,
                                            needs_layout_passes=False),
    )
    return lookup(Tt, xT, cT)


# speedup vs baseline: 29.3317x; 29.3317x over previous
import functools

import numpy as np
import jax
import jax.numpy as jnp
from jax import lax
from jax.experimental import pallas as pl
from jax.experimental.pallas import tpu as pltpu
from jax.experimental.pallas import tpu_sc as plsc

_NL = 16
_T = 2 ** 19
_B = 131072
_P1 = np.int32(np.uint32(2654435761))
_P2 = np.int32(np.uint32(805459861))
_RES = [int(np.floor(16 * 1.4472692012786865 ** l)) for l in range(_NL)]
_DENSE = [(r + 1) ** 3 <= _T for r in _RES]

_NC, _NS = 2, 16
_NW = _NC * _NS
_PW = _B // _NW
_NCH = _PW // 16
_CT = 4096
_CPL = _T // _CT
_PPW = (_NL * _CPL) // _NW


def _mesh():
    return plsc.VectorSubcoreMesh(core_axis_name="c", subcore_axis_name="s")


def _relayout_body(tbl, out, sbuf, buf, sem):
    wid = lax.axis_index("s") * _NC + lax.axis_index("c")
    lane = lax.iota(jnp.int32, 16)
    _TB = _CT // 128

    def body(i, carry):
        piece = wid * _PPW + i
        l = piece // _CPL
        c = piece - l * _CPL
        r0 = c * _CT
        cps = [pltpu.async_copy(tbl.at[h, l, pl.ds(c * _TB, _TB), :, :],
                                sbuf.at[h], sem) for h in range(4)]
        for cp in cps:
            cp.wait()

        def tbody(tb, carry2):
            rbase = tb * 128
            for v in range(8):
                row = rbase + v * 16 + lane
                for h in range(4):
                    for f in range(2):
                        x = sbuf[h, tb, f, pl.ds(v * 16, 16)]
                        plsc.store_scatter(
                            buf, [row, jnp.full((16,), 2 * h + f, jnp.int32)], x)
            return carry2

        lax.fori_loop(0, _TB, tbody, 0)
        pltpu.sync_copy(buf, out.at[pl.ds(l * _T + r0, _CT), :])
        return carry

    lax.fori_loop(0, _PPW, body, 0)


def _lookup_body(Tt, xT, cT, out, xb, cb, idxb, wcb, rows, obuf, gsem):
    wid = lax.axis_index("s") * _NC + lax.axis_index("c")
    base = wid * _PW
    pltpu.sync_copy(xT.at[:, pl.ds(base, _PW)], xb)
    pltpu.sync_copy(cT.at[:, pl.ds(base, _PW)], cb)
    lane = lax.iota(jnp.int32, 16)
    cols = [jnp.full((16,), v, jnp.int32) for v in range(8)]

    def chunk(ch, carry):
        off = ch * 16
        xs = [xb[d, pl.ds(off, 16)] for d in range(3)]
        copies = []
        for l in range(_NL):
            res = _RES[l]
            pos = [xs[d] * jnp.float32(res) for d in range(3)]
            ip = [p.astype(jnp.int32) for p in pos]
            w1 = [pos[d] - ip[d].astype(jnp.float32) for d in range(3)]
            w0 = [jnp.float32(1.0) - w1[d] for d in range(3)]
            c0 = ip
            c1 = [ip[d] + 1 for d in range(3)]
            if _DENSE[l]:
                s = np.int32(res + 1)
                sc2 = [c0[2] * s, c1[2] * s]
                u = [[(c1[1] if b1 else c0[1]) + sc2[b2] for b2 in range(2)]
                     for b1 in range(2)]

                def cidx(b0, b1, b2, s=s, u=u, c0=c0, c1=c1):
                    return (c1[0] if b0 else c0[0]) + s * u[b1][b2]
            else:
                h1 = [c0[1] * _P1, c1[1] * _P1]
                h2 = [c0[2] * _P2, c1[2] * _P2]

                def cidx(b0, b1, b2, h1=h1, h2=h2, c0=c0, c1=c1):
                    h = (c1[0] if b0 else c0[0]) ^ h1[b1] ^ h2[b2]
                    return h & np.int32(_T - 1)

            lbase = np.int32(l * _T)
            p = [[(w1[1] if b1 else w0[1]) * (w1[2] if b2 else w0[2])
                  for b2 in range(2)] for b1 in range(2)]
            for corner in range(8):
                b0, b1, b2 = corner & 1, (corner >> 1) & 1, (corner >> 2) & 1
                o = l * 128 + corner * 16
                idxb[pl.ds(o, 16)] = cidx(b0, b1, b2) + lbase
                wcb[pl.ds(o, 16)] = (w1[0] if b0 else w0[0]) * p[b1][b2]
            copies.append(pltpu.async_copy(
                Tt.at[idxb.at[pl.ds(l * 128, 128)]],
                rows.at[pl.ds(l * 128, 128), :], gsem))
        for cp in copies:
            cp.wait()
        code = [cb[h, pl.ds(off, 16)] for h in range(4)]
        for l in range(_NL):
            rowl = lane + np.int32(l * 128)
            acc = [None, None]
            for corner in range(8):
                rc = rowl + np.int32(corner * 16)
                wc = wcb[pl.ds(l * 128 + corner * 16, 16)]
                for f in range(2):
                    t = code[0] * plsc.load_gather(rows, [rc, cols[f]])
                    for h in range(1, 4):
                        t = t + code[h] * plsc.load_gather(rows, [rc, cols[h * 2 + f]])
                    wt = wc * t
                    acc[f] = wt if acc[f] is None else acc[f] + wt
            for f in range(2):
                plsc.store_scatter(obuf, [lane, jnp.full((16,), 2 * l + f, jnp.int32)],
                                   acc[f])
        pltpu.sync_copy(obuf, out.at[pl.ds(base + off, 16), :])
        return carry

    lax.fori_loop(0, _NCH, chunk, 0)


def kernel(in_tensor, conditioning_code, tables):
    xT = in_tensor.T
    cT = conditioning_code.T
    t5 = jnp.swapaxes(tables.reshape(4, _NL, _T // 128, 128, 2), 3, 4)
    relayout = pl.kernel(
        _relayout_body, mesh=_mesh(),
        out_type=jax.ShapeDtypeStruct((_NL * _T, 8), jnp.float32),
        scratch_types=[
            pltpu.VMEM((4, _CT // 128, 2, 128), jnp.float32),
            pltpu.VMEM((_CT, 8), jnp.float32),
            pltpu.SemaphoreType.DMA,
        ],
        compiler_params=pltpu.CompilerParams(use_tc_tiling_on_sc=False,
                                             needs_layout_passes=False),
    )
    Tt = relayout(t5)
    lookup = pl.kernel(
        _lookup_body, mesh=_mesh(),
        out_type=jax.ShapeDtypeStruct((_B, 32), jnp.float32),
        scratch_types=[
            pltpu.VMEM((3, _PW), jnp.float32),
            pltpu.VMEM((4, _PW), jnp.float32),
            pltpu.VMEM((_NL * 128,), jnp.int32),
            pltpu.VMEM((_NL * 128,), jnp.float32),
            pltpu.VMEM((_NL * 128, 8), jnp.float32),
            pltpu.VMEM((16, 32), jnp.float32),
            pltpu.SemaphoreType.DMA,
        ],
        compiler_params=pltpu.CompilerParams(use_tc_tiling_on_sc=False,
                                            needs_layout_passes=False),
    )
    return lookup(Tt, xT, cT)


# speedup vs baseline: 38.0814x; 1.2983x over previous
import functools

import numpy as np
import jax
import jax.numpy as jnp
from jax import lax
from jax.experimental import pallas as pl
from jax.experimental.pallas import tpu as pltpu
from jax.experimental.pallas import tpu_sc as plsc

_NL = 16
_T = 2 ** 19
_B = 131072
_P1 = np.int32(np.uint32(2654435761))
_P2 = np.int32(np.uint32(805459861))
_RES = [int(np.floor(16 * 1.4472692012786865 ** l)) for l in range(_NL)]
_DENSE = [(r + 1) ** 3 <= _T for r in _RES]

_NC, _NS = 2, 16
_NW = _NC * _NS
_PW = _B // _NW
_NCH = _PW // 16
_CT = 4096
_CPL = _T // _CT
_PPW = (_NL * _CPL) // _NW


def _mesh():
    return plsc.VectorSubcoreMesh(core_axis_name="c", subcore_axis_name="s")


def _relayout_body(tbl, out, sbuf, buf, sem):
    wid = lax.axis_index("s") * _NC + lax.axis_index("c")
    lane = lax.iota(jnp.int32, 16)
    _TB = _CT // 128

    def body(i, carry):
        piece = wid * _PPW + i
        l = piece // _CPL
        c = piece - l * _CPL
        r0 = c * _CT
        cps = [pltpu.async_copy(tbl.at[h, l, pl.ds(c * _TB, _TB), :, :],
                                sbuf.at[h], sem) for h in range(4)]
        for cp in cps:
            cp.wait()

        def tbody(tb, carry2):
            rbase = tb * 128
            for v in range(8):
                row = rbase + v * 16 + lane
                for h in range(4):
                    for f in range(2):
                        x = sbuf[h, tb, f, pl.ds(v * 16, 16)]
                        plsc.store_scatter(
                            buf, [row, jnp.full((16,), 2 * h + f, jnp.int32)], x)
            return carry2

        lax.fori_loop(0, _TB, tbody, 0)
        pltpu.sync_copy(buf, out.at[pl.ds(l * _T + r0, _CT), :])
        return carry

    lax.fori_loop(0, _PPW, body, 0)


def _lookup_body(Tt, xT, cT, out, xb, cb, idxb, wcb, rows, obuf, gsem):
    wid = lax.axis_index("s") * _NC + lax.axis_index("c")
    base = wid * _PW
    pltpu.sync_copy(xT.at[:, pl.ds(base, _PW)], xb)
    pltpu.sync_copy(cT.at[:, pl.ds(base, _PW)], cb)
    lane = lax.iota(jnp.int32, 16)
    cols = [jnp.full((16,), v, jnp.int32) for v in range(8)]

    def idx_phase(ch, poff):
        off = ch * 16
        xs = [xb[d, pl.ds(off, 16)] for d in range(3)]
        for l in range(_NL):
            res = _RES[l]
            pos = [xs[d] * jnp.float32(res) for d in range(3)]
            ip = [p.astype(jnp.int32) for p in pos]
            w1 = [pos[d] - ip[d].astype(jnp.float32) for d in range(3)]
            w0 = [jnp.float32(1.0) - w1[d] for d in range(3)]
            c0 = ip
            c1 = [ip[d] + 1 for d in range(3)]
            if _DENSE[l]:
                s = np.int32(res + 1)
                sc2 = [c0[2] * s, c1[2] * s]
                u = [[(c1[1] if b1 else c0[1]) + sc2[b2] for b2 in range(2)]
                     for b1 in range(2)]

                def cidx(b0, b1, b2, s=s, u=u, c0=c0, c1=c1):
                    return (c1[0] if b0 else c0[0]) + s * u[b1][b2]
            else:
                h1 = [c0[1] * _P1, c1[1] * _P1]
                h2 = [c0[2] * _P2, c1[2] * _P2]

                def cidx(b0, b1, b2, h1=h1, h2=h2, c0=c0, c1=c1):
                    h = (c1[0] if b0 else c0[0]) ^ h1[b1] ^ h2[b2]
                    return h & np.int32(_T - 1)

            lbase = np.int32(l * _T)
            p = [[(w1[1] if b1 else w0[1]) * (w1[2] if b2 else w0[2])
                  for b2 in range(2)] for b1 in range(2)]
            for corner in range(8):
                b0, b1, b2 = corner & 1, (corner >> 1) & 1, (corner >> 2) & 1
                o = l * 128 + corner * 16
                idxb[pl.ds(poff + o, 16)] = cidx(b0, b1, b2) + lbase
                wcb[pl.ds(poff + o, 16)] = (w1[0] if b0 else w0[0]) * p[b1][b2]

    def fire(poff):
        for l in range(_NL):
            pltpu.async_copy(
                Tt.at[idxb.at[pl.ds(poff + l * 128, 128)]],
                rows.at[pl.ds(poff + l * 128, 128), :], gsem)

    def drain(poff):
        for l in range(_NL):
            pltpu.make_async_copy(
                Tt.at[idxb.at[pl.ds(poff + l * 128, 128)]],
                rows.at[pl.ds(poff + l * 128, 128), :], gsem).wait()

    def compute(ch, poff):
        off = ch * 16
        code = [cb[h, pl.ds(off, 16)] for h in range(4)]
        for l in range(_NL):
            rowl = lane + poff + np.int32(l * 128)
            acc = [None, None]
            for corner in range(8):
                rc = rowl + np.int32(corner * 16)
                wc = wcb[pl.ds(poff + l * 128 + corner * 16, 16)]
                for f in range(2):
                    t = code[0] * plsc.load_gather(rows, [rc, cols[f]])
                    for h in range(1, 4):
                        t = t + code[h] * plsc.load_gather(rows, [rc, cols[h * 2 + f]])
                    wt = wc * t
                    acc[f] = wt if acc[f] is None else acc[f] + wt
            for f in range(2):
                plsc.store_scatter(obuf, [lane, jnp.full((16,), 2 * l + f, jnp.int32)],
                                   acc[f])
        pltpu.sync_copy(obuf, out.at[pl.ds(base + off, 16), :])

    idx_phase(0, 0)
    fire(0)

    def chunk(ch, carry):
        poff = (ch & 1) * (_NL * 128)
        npoff = _NL * 128 - poff

        @pl.when(ch + 1 < _NCH)
        def _prefetch():
            idx_phase(ch + 1, npoff)

        drain(poff)

        @pl.when(ch + 1 < _NCH)
        def _fire_next():
            fire(npoff)

        compute(ch, poff)
        return carry

    lax.fori_loop(0, _NCH, chunk, 0)


def kernel(in_tensor, conditioning_code, tables):
    xT = in_tensor.T
    cT = conditioning_code.T
    t5 = jnp.swapaxes(tables.reshape(4, _NL, _T // 128, 128, 2), 3, 4)
    relayout = pl.kernel(
        _relayout_body, mesh=_mesh(),
        out_type=jax.ShapeDtypeStruct((_NL * _T, 8), jnp.float32),
        scratch_types=[
            pltpu.VMEM((4, _CT // 128, 2, 128), jnp.float32),
            pltpu.VMEM((_CT, 8), jnp.float32),
            pltpu.SemaphoreType.DMA,
        ],
        compiler_params=pltpu.CompilerParams(use_tc_tiling_on_sc=False,
                                             needs_layout_passes=False),
    )
    Tt = relayout(t5)
    lookup = pl.kernel(
        _lookup_body, mesh=_mesh(),
        out_type=jax.ShapeDtypeStruct((_B, 32), jnp.float32),
        scratch_types=[
            pltpu.VMEM((3, _PW), jnp.float32),
            pltpu.VMEM((4, _PW), jnp.float32),
            pltpu.VMEM((2 * _NL * 128,), jnp.int32),
            pltpu.VMEM((2 * _NL * 128,), jnp.float32),
            pltpu.VMEM((2 * _NL * 128, 8), jnp.float32),
            pltpu.VMEM((16, 32), jnp.float32),
            pltpu.SemaphoreType.DMA,
        ],
        compiler_params=pltpu.CompilerParams(use_tc_tiling_on_sc=False,
                                            needs_layout_passes=False),
    )
    return lookup(Tt, xT, cT)


# speedup vs baseline: 43.9360x; 1.1537x over previous
import functools

import numpy as np
import jax
import jax.numpy as jnp
from jax import lax
from jax.experimental import pallas as pl
from jax.experimental.pallas import tpu as pltpu
from jax.experimental.pallas import tpu_sc as plsc

_NL = 16
_T = 2 ** 19
_B = 131072
_P1 = np.int32(np.uint32(2654435761))
_P2 = np.int32(np.uint32(805459861))
_RES = [int(np.floor(16 * 1.4472692012786865 ** l)) for l in range(_NL)]
_DENSE = [(r + 1) ** 3 <= _T for r in _RES]

_NC, _NS = 2, 16
_NW = _NC * _NS
_PW = _B // _NW
_NCH = _PW // 16
_CT = 2048
_CPL = _T // _CT
_PPW = (_NL * _CPL) // _NW


def _mesh():
    return plsc.VectorSubcoreMesh(core_axis_name="c", subcore_axis_name="s")


def _relayout_body(tbl, out, sbuf, buf, isem, osem0, osem1):
    wid = lax.axis_index("s") * _NC + lax.axis_index("c")
    lane = lax.iota(jnp.int32, 16)
    _TB = _CT // 128
    colv = [jnp.full((16,), j, jnp.int32) for j in range(8)]

    def piece_lc(i):
        piece = wid * _PPW + i
        l = piece // _CPL
        c = piece - l * _CPL
        return l, c

    def fire_in(i, par):
        l, c = piece_lc(i)
        for h in range(4):
            pltpu.async_copy(tbl.at[h, l, pl.ds(c * _TB, _TB), :, :],
                             sbuf.at[par, h], isem)

    def drain_in(i, par):
        l, c = piece_lc(i)
        for h in range(4):
            pltpu.make_async_copy(tbl.at[h, l, pl.ds(c * _TB, _TB), :, :],
                                  sbuf.at[par, h], isem).wait()

    def interleave(par):
        def tbody(tb, carry2):
            rbase = tb * 128
            for v in range(8):
                row = rbase + v * 16 + lane
                for h in range(4):
                    for f in range(2):
                        x = sbuf[par, h, tb, f, pl.ds(v * 16, 16)]
                        plsc.store_scatter(buf.at[par], [row, colv[2 * h + f]], x)
            return carry2

        lax.fori_loop(0, _TB, tbody, 0)

    def out_ref(i):
        l, c = piece_lc(i)
        return out.at[pl.ds(l * _T + c * _CT, _CT), :]

    def fire_out(i, par, sem):
        pltpu.async_copy(buf.at[par], out_ref(i), sem)

    def drain_out(i, par, sem):
        pltpu.make_async_copy(buf.at[par], out_ref(i), sem).wait()

    fire_in(0, 0)

    def body(j, carry):
        i0 = 2 * j
        drain_in(i0, 0)
        fire_in(i0 + 1, 1)

        @pl.when(j > 0)
        def _():
            drain_out(i0 - 2, 0, osem0)

        interleave(0)
        fire_out(i0, 0, osem0)

        drain_in(i0 + 1, 1)

        @pl.when(j + 1 < _PPW // 2)
        def _():
            fire_in(i0 + 2, 0)

        @pl.when(j > 0)
        def _():
            drain_out(i0 - 1, 1, osem1)

        interleave(1)
        fire_out(i0 + 1, 1, osem1)
        return carry

    lax.fori_loop(0, _PPW // 2, body, 0)
    drain_out(_PPW - 2, 0, osem0)
    drain_out(_PPW - 1, 1, osem1)


def _lookup_body(Tt, xT, cT, out, xb, cb, idxb, wcb, rows, obuf, gsem):
    wid = lax.axis_index("s") * _NC + lax.axis_index("c")
    base = wid * _PW
    pltpu.sync_copy(xT.at[:, pl.ds(base, _PW)], xb)
    pltpu.sync_copy(cT.at[:, pl.ds(base, _PW)], cb)
    lane = lax.iota(jnp.int32, 16)
    cols = [jnp.full((16,), v, jnp.int32) for v in range(8)]

    def idx_phase(ch, poff):
        off = ch * 16
        xs = [xb[d, pl.ds(off, 16)] for d in range(3)]
        for l in range(_NL):
            res = _RES[l]
            pos = [xs[d] * jnp.float32(res) for d in range(3)]
            ip = [p.astype(jnp.int32) for p in pos]
            w1 = [pos[d] - ip[d].astype(jnp.float32) for d in range(3)]
            w0 = [jnp.float32(1.0) - w1[d] for d in range(3)]
            c0 = ip
            c1 = [ip[d] + 1 for d in range(3)]
            if _DENSE[l]:
                s = np.int32(res + 1)
                sc2 = [c0[2] * s, c1[2] * s]
                u = [[(c1[1] if b1 else c0[1]) + sc2[b2] for b2 in range(2)]
                     for b1 in range(2)]

                def cidx(b0, b1, b2, s=s, u=u, c0=c0, c1=c1):
                    return (c1[0] if b0 else c0[0]) + s * u[b1][b2]
            else:
                h1 = [c0[1] * _P1, c1[1] * _P1]
                h2 = [c0[2] * _P2, c1[2] * _P2]

                def cidx(b0, b1, b2, h1=h1, h2=h2, c0=c0, c1=c1):
                    h = (c1[0] if b0 else c0[0]) ^ h1[b1] ^ h2[b2]
                    return h & np.int32(_T - 1)

            lbase = np.int32(l * _T)
            p = [[(w1[1] if b1 else w0[1]) * (w1[2] if b2 else w0[2])
                  for b2 in range(2)] for b1 in range(2)]
            for corner in range(8):
                b0, b1, b2 = corner & 1, (corner >> 1) & 1, (corner >> 2) & 1
                o = l * 128 + corner * 16
                idxb[pl.ds(poff + o, 16)] = cidx(b0, b1, b2) + lbase
                wcb[pl.ds(poff + o, 16)] = (w1[0] if b0 else w0[0]) * p[b1][b2]

    def fire(poff):
        for l in range(_NL):
            pltpu.async_copy(
                Tt.at[idxb.at[pl.ds(poff + l * 128, 128)]],
                rows.at[pl.ds(poff + l * 128, 128), :], gsem)

    def drain(poff):
        for l in range(_NL):
            pltpu.make_async_copy(
                Tt.at[idxb.at[pl.ds(poff + l * 128, 128)]],
                rows.at[pl.ds(poff + l * 128, 128), :], gsem).wait()

    def compute(ch, poff):
        off = ch * 16
        code = [cb[h, pl.ds(off, 16)] for h in range(4)]
        for l in range(_NL):
            rowl = lane + poff + np.int32(l * 128)
            acc = [None, None]
            for corner in range(8):
                rc = rowl + np.int32(corner * 16)
                wc = wcb[pl.ds(poff + l * 128 + corner * 16, 16)]
                for f in range(2):
                    t = code[0] * plsc.load_gather(rows, [rc, cols[f]])
                    for h in range(1, 4):
                        t = t + code[h] * plsc.load_gather(rows, [rc, cols[h * 2 + f]])
                    wt = wc * t
                    acc[f] = wt if acc[f] is None else acc[f] + wt
            for f in range(2):
                plsc.store_scatter(obuf, [lane, jnp.full((16,), 2 * l + f, jnp.int32)],
                                   acc[f])
        pltpu.sync_copy(obuf, out.at[pl.ds(base + off, 16), :])

    idx_phase(0, 0)
    fire(0)

    def chunk(ch, carry):
        poff = (ch & 1) * (_NL * 128)
        npoff = _NL * 128 - poff

        @pl.when(ch + 1 < _NCH)
        def _prefetch():
            idx_phase(ch + 1, npoff)

        drain(poff)

        @pl.when(ch + 1 < _NCH)
        def _fire_next():
            fire(npoff)

        compute(ch, poff)
        return carry

    lax.fori_loop(0, _NCH, chunk, 0)


def kernel(in_tensor, conditioning_code, tables):
    xT = in_tensor.T
    cT = conditioning_code.T
    t5 = jnp.swapaxes(tables.reshape(4, _NL, _T // 128, 128, 2), 3, 4)
    relayout = pl.kernel(
        _relayout_body, mesh=_mesh(),
        out_type=jax.ShapeDtypeStruct((_NL * _T, 8), jnp.float32),
        scratch_types=[
            pltpu.VMEM((2, 4, _CT // 128, 2, 128), jnp.float32),
            pltpu.VMEM((2, _CT, 8), jnp.float32),
            pltpu.SemaphoreType.DMA,
            pltpu.SemaphoreType.DMA,
            pltpu.SemaphoreType.DMA,
        ],
        compiler_params=pltpu.CompilerParams(use_tc_tiling_on_sc=False,
                                             needs_layout_passes=False),
    )
    Tt = relayout(t5)
    lookup = pl.kernel(
        _lookup_body, mesh=_mesh(),
        out_type=jax.ShapeDtypeStruct((_B, 32), jnp.float32),
        scratch_types=[
            pltpu.VMEM((3, _PW), jnp.float32),
            pltpu.VMEM((4, _PW), jnp.float32),
            pltpu.VMEM((2 * _NL * 128,), jnp.int32),
            pltpu.VMEM((2 * _NL * 128,), jnp.float32),
            pltpu.VMEM((2 * _NL * 128, 8), jnp.float32),
            pltpu.VMEM((16, 32), jnp.float32),
            pltpu.SemaphoreType.DMA,
        ],
        compiler_params=pltpu.CompilerParams(use_tc_tiling_on_sc=False,
                                            needs_layout_passes=False),
    )
    return lookup(Tt, xT, cT)


# speedup vs baseline: 44.0831x; 1.0033x over previous
import functools

import numpy as np
import jax
import jax.numpy as jnp
from jax import lax
from jax.experimental import pallas as pl
from jax.experimental.pallas import tpu as pltpu
from jax.experimental.pallas import tpu_sc as plsc

_NL = 16
_T = 2 ** 19
_B = 131072
_P1 = np.int32(np.uint32(2654435761))
_P2 = np.int32(np.uint32(805459861))
_RES = [int(np.floor(16 * 1.4472692012786865 ** l)) for l in range(_NL)]
_DENSE = [(r + 1) ** 3 <= _T for r in _RES]

_NC, _NS = 2, 16
_NW = _NC * _NS
_PW = _B // _NW
_NCH = _PW // 16
_CT = 2048
_CPL = _T // _CT
_PPW = (_NL * _CPL) // _NW


def _mesh():
    return plsc.VectorSubcoreMesh(core_axis_name="c", subcore_axis_name="s")


def _relayout_body(tbl, out, sbuf, buf, isem, osem0, osem1):
    wid = lax.axis_index("s") * _NC + lax.axis_index("c")
    lane = lax.iota(jnp.int32, 16)
    _TB = _CT // 128
    colv = [jnp.full((16,), j, jnp.int32) for j in range(8)]

    def piece_lc(i):
        piece = wid * _PPW + i
        l = piece // _CPL
        c = piece - l * _CPL
        return l, c

    def fire_in(i, par):
        l, c = piece_lc(i)
        for h in range(4):
            pltpu.async_copy(tbl.at[h, l, pl.ds(c * _TB, _TB), :, :],
                             sbuf.at[par, h], isem)

    def drain_in(i, par):
        l, c = piece_lc(i)
        for h in range(4):
            pltpu.make_async_copy(tbl.at[h, l, pl.ds(c * _TB, _TB), :, :],
                                  sbuf.at[par, h], isem).wait()

    def interleave(par):
        def tbody(tb, carry2):
            rbase = tb * 128
            for v in range(8):
                row = rbase + v * 16 + lane
                for h in range(4):
                    for f in range(2):
                        x = sbuf[par, h, tb, f, pl.ds(v * 16, 16)]
                        plsc.store_scatter(buf.at[par], [row, colv[2 * h + f]], x)
            return carry2

        lax.fori_loop(0, _TB, tbody, 0)

    def out_ref(i):
        l, c = piece_lc(i)
        return out.at[pl.ds(l * _T + c * _CT, _CT), :]

    def fire_out(i, par, sem):
        pltpu.async_copy(buf.at[par], out_ref(i), sem)

    def drain_out(i, par, sem):
        pltpu.make_async_copy(buf.at[par], out_ref(i), sem).wait()

    fire_in(0, 0)

    def body(j, carry):
        i0 = 2 * j
        drain_in(i0, 0)
        fire_in(i0 + 1, 1)

        @pl.when(j > 0)
        def _():
            drain_out(i0 - 2, 0, osem0)

        interleave(0)
        fire_out(i0, 0, osem0)

        drain_in(i0 + 1, 1)

        @pl.when(j + 1 < _PPW // 2)
        def _():
            fire_in(i0 + 2, 0)

        @pl.when(j > 0)
        def _():
            drain_out(i0 - 1, 1, osem1)

        interleave(1)
        fire_out(i0 + 1, 1, osem1)
        return carry

    lax.fori_loop(0, _PPW // 2, body, 0)
    drain_out(_PPW - 2, 0, osem0)
    drain_out(_PPW - 1, 1, osem1)


def _lookup_body(Tt, xT, cT, out, xb, cb, idxb, wcb, rows, obuf, gsem):
    wid = lax.axis_index("s") * _NC + lax.axis_index("c")
    base = wid * _PW
    pltpu.sync_copy(xT.at[:, pl.ds(base, _PW)], xb)
    pltpu.sync_copy(cT.at[:, pl.ds(base, _PW)], cb)
    lane = lax.iota(jnp.int32, 16)
    cols = [jnp.full((16,), v, jnp.int32) for v in range(8)]

    def idx_phase(ch, poff):
        off = ch * 16
        xs = [xb[d, pl.ds(off, 16)] for d in range(3)]
        for l in range(_NL):
            res = _RES[l]
            pos = [xs[d] * jnp.float32(res) for d in range(3)]
            ip = [p.astype(jnp.int32) for p in pos]
            w1 = [pos[d] - ip[d].astype(jnp.float32) for d in range(3)]
            w0 = [jnp.float32(1.0) - w1[d] for d in range(3)]
            c0 = ip
            c1 = [ip[d] + 1 for d in range(3)]
            if _DENSE[l]:
                s = np.int32(res + 1)
                sc2 = [c0[2] * s, c1[2] * s]
                u = [[(c1[1] if b1 else c0[1]) + sc2[b2] for b2 in range(2)]
                     for b1 in range(2)]

                def cidx(b0, b1, b2, s=s, u=u, c0=c0, c1=c1):
                    return (c1[0] if b0 else c0[0]) + s * u[b1][b2]
            else:
                m = np.int32(_T - 1)
                g0 = [c0[0] & m, c1[0] & m]
                h1 = [(c0[1] * _P1) & m, (c1[1] * _P1) & m]
                h2 = [(c0[2] * _P2) & m, (c1[2] * _P2) & m]

                def cidx(b0, b1, b2, g0=g0, h1=h1, h2=h2):
                    return g0[b0] ^ h1[b1] ^ h2[b2]

            lbase = np.int32(l * _T)
            p = [[(w1[1] if b1 else w0[1]) * (w1[2] if b2 else w0[2])
                  for b2 in range(2)] for b1 in range(2)]
            for corner in range(8):
                b0, b1, b2 = corner & 1, (corner >> 1) & 1, (corner >> 2) & 1
                o = l * 128 + corner * 16
                idxb[pl.ds(poff + o, 16)] = cidx(b0, b1, b2) + lbase
                wcb[pl.ds(poff + o, 16)] = (w1[0] if b0 else w0[0]) * p[b1][b2]

    def fire(poff):
        for l in range(_NL):
            pltpu.async_copy(
                Tt.at[idxb.at[pl.ds(poff + l * 128, 128)]],
                rows.at[pl.ds(poff + l * 128, 128), :], gsem)

    def drain(poff):
        for l in range(_NL):
            pltpu.make_async_copy(
                Tt.at[idxb.at[pl.ds(poff + l * 128, 128)]],
                rows.at[pl.ds(poff + l * 128, 128), :], gsem).wait()

    def compute(ch, poff):
        off = ch * 16
        code = [cb[h, pl.ds(off, 16)] for h in range(4)]
        for l in range(_NL):
            acc = [None, None]
            for corner in range(8):
                o = poff + l * 128 + corner * 16
                rslice = rows.at[pl.ds(o, 16), :]
                wc = wcb[pl.ds(o, 16)]
                for f in range(2):
                    t = code[0] * plsc.load_gather(rslice, [lane, cols[f]])
                    for h in range(1, 4):
                        t = t + code[h] * plsc.load_gather(rslice, [lane, cols[h * 2 + f]])
                    wt = wc * t
                    acc[f] = wt if acc[f] is None else acc[f] + wt
            for f in range(2):
                plsc.store_scatter(obuf, [lane, jnp.full((16,), 2 * l + f, jnp.int32)],
                                   acc[f])
        pltpu.sync_copy(obuf, out.at[pl.ds(base + off, 16), :])

    idx_phase(0, 0)
    fire(0)

    def chunk(ch, carry):
        poff = (ch & 1) * (_NL * 128)
        npoff = _NL * 128 - poff

        @pl.when(ch + 1 < _NCH)
        def _prefetch():
            idx_phase(ch + 1, npoff)

        drain(poff)

        @pl.when(ch + 1 < _NCH)
        def _fire_next():
            fire(npoff)

        compute(ch, poff)
        return carry

    lax.fori_loop(0, _NCH, chunk, 0)


def kernel(in_tensor, conditioning_code, tables):
    xT = in_tensor.T
    cT = conditioning_code.T
    t5 = jnp.swapaxes(tables.reshape(4, _NL, _T // 128, 128, 2), 3, 4)
    relayout = pl.kernel(
        _relayout_body, mesh=_mesh(),
        out_type=jax.ShapeDtypeStruct((_NL * _T, 8), jnp.float32),
        scratch_types=[
            pltpu.VMEM((2, 4, _CT // 128, 2, 128), jnp.float32),
            pltpu.VMEM((2, _CT, 8), jnp.float32),
            pltpu.SemaphoreType.DMA,
            pltpu.SemaphoreType.DMA,
            pltpu.SemaphoreType.DMA,
        ],
        compiler_params=pltpu.CompilerParams(use_tc_tiling_on_sc=False,
                                             needs_layout_passes=False),
    )
    Tt = relayout(t5)
    lookup = pl.kernel(
        _lookup_body, mesh=_mesh(),
        out_type=jax.ShapeDtypeStruct((_B, 32), jnp.float32),
        scratch_types=[
            pltpu.VMEM((3, _PW), jnp.float32),
            pltpu.VMEM((4, _PW), jnp.float32),
            pltpu.VMEM((2 * _NL * 128,), jnp.int32),
            pltpu.VMEM((2 * _NL * 128,), jnp.float32),
            pltpu.VMEM((2 * _NL * 128, 8), jnp.float32),
            pltpu.VMEM((16, 32), jnp.float32),
            pltpu.SemaphoreType.DMA,
        ],
        compiler_params=pltpu.CompilerParams(use_tc_tiling_on_sc=False,
                                            needs_layout_passes=False),
    )
    return lookup(Tt, xT, cT)


# speedup vs baseline: 44.1774x; 1.0021x over previous
import functools

import numpy as np
import jax
import jax.numpy as jnp
from jax import lax
from jax.experimental import pallas as pl
from jax.experimental.pallas import tpu as pltpu
from jax.experimental.pallas import tpu_sc as plsc

_NL = 16
_T = 2 ** 19
_B = 131072
_P1 = np.int32(np.uint32(2654435761))
_P2 = np.int32(np.uint32(805459861))
_RES = [int(np.floor(16 * 1.4472692012786865 ** l)) for l in range(_NL)]
_DENSE = [(r + 1) ** 3 <= _T for r in _RES]

_NC, _NS = 2, 16
_NW = _NC * _NS
_PW = _B // _NW
_NCH = _PW // 16
_CT = 2048
_CPL = _T // _CT
_PPW = (_NL * _CPL) // _NW


def _mesh():
    return plsc.VectorSubcoreMesh(core_axis_name="c", subcore_axis_name="s")


def _relayout_body(tbl, out, sbuf, buf, isem, osem0, osem1):
    wid = lax.axis_index("s") * _NC + lax.axis_index("c")
    lane = lax.iota(jnp.int32, 16)
    _TB = _CT // 128
    colv = [jnp.full((16,), j, jnp.int32) for j in range(8)]

    def piece_lc(i):
        piece = wid * _PPW + i
        l = piece // _CPL
        c = piece - l * _CPL
        return l, c

    def fire_in(i, par):
        l, c = piece_lc(i)
        for h in range(4):
            pltpu.async_copy(tbl.at[h, l, pl.ds(c * _TB, _TB), :, :],
                             sbuf.at[par, h], isem)

    def drain_in(i, par):
        l, c = piece_lc(i)
        for h in range(4):
            pltpu.make_async_copy(tbl.at[h, l, pl.ds(c * _TB, _TB), :, :],
                                  sbuf.at[par, h], isem).wait()

    def interleave(par):
        def tbody(tb, carry2):
            rbase = tb * 128
            for v in range(8):
                row = rbase + v * 16 + lane
                for h in range(4):
                    for f in range(2):
                        x = sbuf[par, h, tb, f, pl.ds(v * 16, 16)]
                        plsc.store_scatter(buf.at[par], [row, colv[2 * h + f]], x)
            return carry2

        lax.fori_loop(0, _TB, tbody, 0)

    def out_ref(i):
        l, c = piece_lc(i)
        return out.at[pl.ds(l * _T + c * _CT, _CT), :]

    def fire_out(i, par, sem):
        pltpu.async_copy(buf.at[par], out_ref(i), sem)

    def drain_out(i, par, sem):
        pltpu.make_async_copy(buf.at[par], out_ref(i), sem).wait()

    fire_in(0, 0)

    def body(j, carry):
        i0 = 2 * j
        drain_in(i0, 0)
        fire_in(i0 + 1, 1)

        @pl.when(j > 0)
        def _():
            drain_out(i0 - 2, 0, osem0)

        interleave(0)
        fire_out(i0, 0, osem0)

        drain_in(i0 + 1, 1)

        @pl.when(j + 1 < _PPW // 2)
        def _():
            fire_in(i0 + 2, 0)

        @pl.when(j > 0)
        def _():
            drain_out(i0 - 1, 1, osem1)

        interleave(1)
        fire_out(i0 + 1, 1, osem1)
        return carry

    lax.fori_loop(0, _PPW // 2, body, 0)
    drain_out(_PPW - 2, 0, osem0)
    drain_out(_PPW - 1, 1, osem1)


def _lookup_body(Tt, xT, cT, out, xb, cb, idxb, wcb, rows, obuf, gsem):
    wid = lax.axis_index("s") * _NC + lax.axis_index("c")
    base = wid * _PW
    pltpu.sync_copy(xT.at[:, pl.ds(base, _PW)], xb.at[:, pl.ds(0, _PW)])
    pltpu.sync_copy(cT.at[:, pl.ds(base, _PW)], cb)
    lane = lax.iota(jnp.int32, 16)
    cols = [jnp.full((16,), v, jnp.int32) for v in range(8)]

    def idx_phase(ch, poff):
        off = ch * 16
        xs = [xb[d, pl.ds(off, 16)] for d in range(3)]
        for l in range(_NL):
            res = _RES[l]
            pos = [xs[d] * jnp.float32(res) for d in range(3)]
            ip = [p.astype(jnp.int32) for p in pos]
            w1 = [pos[d] - ip[d].astype(jnp.float32) for d in range(3)]
            w0 = [jnp.float32(1.0) - w1[d] for d in range(3)]
            c0 = ip
            c1 = [ip[d] + 1 for d in range(3)]
            if _DENSE[l]:
                s = np.int32(res + 1)
                m = np.int32(_T - 1)
                sc2 = [c0[2] * s, c1[2] * s]
                u = [[(c1[1] if b1 else c0[1]) + sc2[b2] for b2 in range(2)]
                     for b1 in range(2)]

                def cidx(b0, b1, b2, s=s, u=u, c0=c0, c1=c1, m=m):
                    return ((c1[0] if b0 else c0[0]) + s * u[b1][b2]) & m
            else:
                m = np.int32(_T - 1)
                g0 = [c0[0] & m, c1[0] & m]
                h1 = [(c0[1] * _P1) & m, (c1[1] * _P1) & m]
                h2 = [(c0[2] * _P2) & m, (c1[2] * _P2) & m]

                def cidx(b0, b1, b2, g0=g0, h1=h1, h2=h2):
                    return g0[b0] ^ h1[b1] ^ h2[b2]

            lbase = np.int32(l * _T)
            p = [[(w1[1] if b1 else w0[1]) * (w1[2] if b2 else w0[2])
                  for b2 in range(2)] for b1 in range(2)]
            for corner in range(8):
                b0, b1, b2 = corner & 1, (corner >> 1) & 1, (corner >> 2) & 1
                o = l * 128 + corner * 16
                idxb[pl.ds(poff + o, 16)] = cidx(b0, b1, b2) + lbase
                wcb[pl.ds(poff + o, 16)] = (w1[0] if b0 else w0[0]) * p[b1][b2]

    def fire(poff):
        for l in range(_NL):
            pltpu.async_copy(
                Tt.at[idxb.at[pl.ds(poff + l * 128, 128)]],
                rows.at[pl.ds(poff + l * 128, 128), :], gsem)

    def drain(poff):
        for l in range(_NL):
            pltpu.make_async_copy(
                Tt.at[idxb.at[pl.ds(poff + l * 128, 128)]],
                rows.at[pl.ds(poff + l * 128, 128), :], gsem).wait()

    def compute(ch, poff):
        off = ch * 16
        code = [cb[h, pl.ds(off, 16)] for h in range(4)]
        for l in range(_NL):
            acc = [None, None]
            for corner in range(8):
                o = poff + l * 128 + corner * 16
                rslice = rows.at[pl.ds(o, 16), :]
                wc = wcb[pl.ds(o, 16)]
                for f in range(2):
                    t = code[0] * plsc.load_gather(rslice, [lane, cols[f]])
                    for h in range(1, 4):
                        t = t + code[h] * plsc.load_gather(rslice, [lane, cols[h * 2 + f]])
                    wt = wc * t
                    acc[f] = wt if acc[f] is None else acc[f] + wt
            for f in range(2):
                plsc.store_scatter(obuf, [lane, jnp.full((16,), 2 * l + f, jnp.int32)],
                                   acc[f])
        pltpu.sync_copy(obuf, out.at[pl.ds(base + off, 16), :])

    idx_phase(0, 0)
    fire(0)

    def chunk(ch, carry):
        poff = (ch & 1) * (_NL * 128)
        npoff = _NL * 128 - poff
        idx_phase(ch + 1, npoff)
        drain(poff)
        fire(npoff)
        compute(ch, poff)
        return carry

    lax.fori_loop(0, _NCH, chunk, 0)
    drain(0)


def kernel(in_tensor, conditioning_code, tables):
    xT = in_tensor.T
    cT = conditioning_code.T
    t5 = jnp.swapaxes(tables.reshape(4, _NL, _T // 128, 128, 2), 3, 4)
    relayout = pl.kernel(
        _relayout_body, mesh=_mesh(),
        out_type=jax.ShapeDtypeStruct((_NL * _T, 8), jnp.float32),
        scratch_types=[
            pltpu.VMEM((2, 4, _CT // 128, 2, 128), jnp.float32),
            pltpu.VMEM((2, _CT, 8), jnp.float32),
            pltpu.SemaphoreType.DMA,
            pltpu.SemaphoreType.DMA,
            pltpu.SemaphoreType.DMA,
        ],
        compiler_params=pltpu.CompilerParams(use_tc_tiling_on_sc=False,
                                             needs_layout_passes=False),
    )
    Tt = relayout(t5)
    lookup = pl.kernel(
        _lookup_body, mesh=_mesh(),
        out_type=jax.ShapeDtypeStruct((_B, 32), jnp.float32),
        scratch_types=[
            pltpu.VMEM((3, _PW + 16), jnp.float32),
            pltpu.VMEM((4, _PW), jnp.float32),
            pltpu.VMEM((2 * _NL * 128,), jnp.int32),
            pltpu.VMEM((2 * _NL * 128,), jnp.float32),
            pltpu.VMEM((2 * _NL * 128, 8), jnp.float32),
            pltpu.VMEM((16, 32), jnp.float32),
            pltpu.SemaphoreType.DMA,
        ],
        compiler_params=pltpu.CompilerParams(use_tc_tiling_on_sc=False,
                                            needs_layout_passes=False),
    )
    return lookup(Tt, xT, cT)


# speedup vs baseline: 44.3396x; 1.0037x over previous
import functools

import numpy as np
import jax
import jax.numpy as jnp
from jax import lax
from jax.experimental import pallas as pl
from jax.experimental.pallas import tpu as pltpu
from jax.experimental.pallas import tpu_sc as plsc

_NL = 16
_T = 2 ** 19
_B = 131072
_P1 = np.int32(np.uint32(2654435761))
_P2 = np.int32(np.uint32(805459861))
_RES = [int(np.floor(16 * 1.4472692012786865 ** l)) for l in range(_NL)]
_DENSE = [(r + 1) ** 3 <= _T for r in _RES]

_NC, _NS = 2, 16
_NW = _NC * _NS
_PW = _B // _NW
_NCH = _PW // 16
_CT = 2048
_CPL = _T // _CT
_PPW = (_NL * _CPL) // _NW


def _mesh():
    return plsc.VectorSubcoreMesh(core_axis_name="c", subcore_axis_name="s")


def _relayout_body(tbl, out, sbuf, buf, isem, osem0, osem1):
    wid = lax.axis_index("s") * _NC + lax.axis_index("c")
    lane = lax.iota(jnp.int32, 16)
    _TB = _CT // 128
    colv = [jnp.full((16,), j, jnp.int32) for j in range(8)]

    def piece_lc(i):
        piece = wid * _PPW + i
        l = piece // _CPL
        c = piece - l * _CPL
        return l, c

    def fire_in(i, par):
        l, c = piece_lc(i)
        for h in range(4):
            pltpu.async_copy(tbl.at[h, l, pl.ds(c * _TB, _TB), :, :],
                             sbuf.at[par, h], isem)

    def drain_in(i, par):
        l, c = piece_lc(i)
        for h in range(4):
            pltpu.make_async_copy(tbl.at[h, l, pl.ds(c * _TB, _TB), :, :],
                                  sbuf.at[par, h], isem).wait()

    def interleave(par):
        def tbody(tb, carry2):
            rbase = tb * 128
            for v in range(8):
                row = rbase + v * 16 + lane
                for h in range(4):
                    for f in range(2):
                        x = sbuf[par, h, tb, f, pl.ds(v * 16, 16)]
                        plsc.store_scatter(buf.at[par], [row, colv[2 * h + f]], x)
            return carry2

        lax.fori_loop(0, _TB, tbody, 0)

    def out_ref(i):
        l, c = piece_lc(i)
        return out.at[pl.ds(l * _T + c * _CT, _CT), :]

    def fire_out(i, par, sem):
        pltpu.async_copy(buf.at[par], out_ref(i), sem)

    def drain_out(i, par, sem):
        pltpu.make_async_copy(buf.at[par], out_ref(i), sem).wait()

    fire_in(0, 0)

    def body(j, carry):
        i0 = 2 * j
        drain_in(i0, 0)
        fire_in(i0 + 1, 1)

        @pl.when(j > 0)
        def _():
            drain_out(i0 - 2, 0, osem0)

        interleave(0)
        fire_out(i0, 0, osem0)

        drain_in(i0 + 1, 1)

        @pl.when(j + 1 < _PPW // 2)
        def _():
            fire_in(i0 + 2, 0)

        @pl.when(j > 0)
        def _():
            drain_out(i0 - 1, 1, osem1)

        interleave(1)
        fire_out(i0 + 1, 1, osem1)
        return carry

    lax.fori_loop(0, _PPW // 2, body, 0)
    drain_out(_PPW - 2, 0, osem0)
    drain_out(_PPW - 1, 1, osem1)


def _lookup_body(Tt, xT, cT, out, xb, cb, idxb, wcb, rows, obuf, gsem, osem):
    wid = lax.axis_index("s") * _NC + lax.axis_index("c")
    base = wid * _PW
    pltpu.sync_copy(xT.at[:, pl.ds(base, _PW)], xb.at[:, pl.ds(0, _PW)])
    pltpu.sync_copy(cT.at[:, pl.ds(base, _PW)], cb)
    lane = lax.iota(jnp.int32, 16)
    cols = [jnp.full((16,), v, jnp.int32) for v in range(8)]

    def idx_phase(ch, poff):
        off = ch * 16
        xs = [xb[d, pl.ds(off, 16)] for d in range(3)]
        for l in range(_NL):
            res = _RES[l]
            pos = [xs[d] * jnp.float32(res) for d in range(3)]
            ip = [p.astype(jnp.int32) for p in pos]
            w1 = [pos[d] - ip[d].astype(jnp.float32) for d in range(3)]
            w0 = [jnp.float32(1.0) - w1[d] for d in range(3)]
            c0 = ip
            c1 = [ip[d] + 1 for d in range(3)]
            if _DENSE[l]:
                s = np.int32(res + 1)
                m = np.int32(_T - 1)
                sc2 = [c0[2] * s, c1[2] * s]
                u = [[(c1[1] if b1 else c0[1]) + sc2[b2] for b2 in range(2)]
                     for b1 in range(2)]

                def cidx(b0, b1, b2, s=s, u=u, c0=c0, c1=c1, m=m):
                    return ((c1[0] if b0 else c0[0]) + s * u[b1][b2]) & m
            else:
                m = np.int32(_T - 1)
                g0 = [c0[0] & m, c1[0] & m]
                h1 = [(c0[1] * _P1) & m, (c1[1] * _P1) & m]
                h2 = [(c0[2] * _P2) & m, (c1[2] * _P2) & m]

                def cidx(b0, b1, b2, g0=g0, h1=h1, h2=h2):
                    return g0[b0] ^ h1[b1] ^ h2[b2]

            lbase = np.int32(l * _T)
            p = [[(w1[1] if b1 else w0[1]) * (w1[2] if b2 else w0[2])
                  for b2 in range(2)] for b1 in range(2)]
            for corner in range(8):
                b0, b1, b2 = corner & 1, (corner >> 1) & 1, (corner >> 2) & 1
                o = l * 128 + corner * 16
                idxb[pl.ds(poff + o, 16)] = cidx(b0, b1, b2) + lbase
                wcb[pl.ds(poff + o, 16)] = (w1[0] if b0 else w0[0]) * p[b1][b2]

    def fire(poff):
        for l in range(_NL):
            pltpu.async_copy(
                Tt.at[idxb.at[pl.ds(poff + l * 128, 128)]],
                rows.at[pl.ds(poff + l * 128, 128), :], gsem)

    def drain(poff):
        pltpu.make_async_copy(
            Tt.at[pl.ds(0, _NL * 128), :],
            rows.at[pl.ds(poff, _NL * 128), :], gsem).wait()

    ocols = [jnp.full((16,), e, jnp.int32) for e in range(32)]

    def compute(ch, poff):
        off = ch * 16
        code = [cb[h, pl.ds(off, 16)] for h in range(4)]
        orow = lane + ((ch & 15) << 4)
        for l in range(_NL):
            acc = [None, None]
            for corner in range(8):
                o = poff + l * 128 + corner * 16
                rslice = rows.at[pl.ds(o, 16), :]
                wc = wcb[pl.ds(o, 16)]
                for f in range(2):
                    t = code[0] * plsc.load_gather(rslice, [lane, cols[f]])
                    for h in range(1, 4):
                        t = t + code[h] * plsc.load_gather(rslice, [lane, cols[h * 2 + f]])
                    wt = wc * t
                    acc[f] = wt if acc[f] is None else acc[f] + wt
            for f in range(2):
                plsc.store_scatter(obuf, [orow, ocols[2 * l + f]], acc[f])

    idx_phase(0, 0)
    fire(0)

    def chunk(ch, carry):
        poff = (ch & 1) * (_NL * 128)
        npoff = _NL * 128 - poff
        half = (ch >> 3) & 1

        @pl.when(jnp.logical_and(ch & 7 == 0, ch >= 16))
        def _drain_out():
            pltpu.make_async_copy(
                obuf.at[pl.ds(half * 128, 128), :],
                out.at[pl.ds(base + (ch - 16) * 16, 128), :], osem).wait()

        idx_phase(ch + 1, npoff)
        drain(poff)
        fire(npoff)
        compute(ch, poff)

        @pl.when(ch & 7 == 7)
        def _fire_out():
            pltpu.async_copy(
                obuf.at[pl.ds(half * 128, 128), :],
                out.at[pl.ds(base + (ch - 7) * 16, 128), :], osem)

        return carry

    lax.fori_loop(0, _NCH, chunk, 0)
    drain(0)
    pltpu.make_async_copy(
        obuf.at[pl.ds(0, 128), :],
        out.at[pl.ds(base + (_NCH - 16) * 16, 128), :], osem).wait()
    pltpu.make_async_copy(
        obuf.at[pl.ds(128, 128), :],
        out.at[pl.ds(base + (_NCH - 8) * 16, 128), :], osem).wait()


def kernel(in_tensor, conditioning_code, tables):
    xT = in_tensor.T
    cT = conditioning_code.T
    t5 = jnp.swapaxes(tables.reshape(4, _NL, _T // 128, 128, 2), 3, 4)
    relayout = pl.kernel(
        _relayout_body, mesh=_mesh(),
        out_type=jax.ShapeDtypeStruct((_NL * _T, 8), jnp.float32),
        scratch_types=[
            pltpu.VMEM((2, 4, _CT // 128, 2, 128), jnp.float32),
            pltpu.VMEM((2, _CT, 8), jnp.float32),
            pltpu.SemaphoreType.DMA,
            pltpu.SemaphoreType.DMA,
            pltpu.SemaphoreType.DMA,
        ],
        compiler_params=pltpu.CompilerParams(use_tc_tiling_on_sc=False,
                                             needs_layout_passes=False),
    )
    Tt = relayout(t5)
    lookup = pl.kernel(
        _lookup_body, mesh=_mesh(),
        out_type=jax.ShapeDtypeStruct((_B, 32), jnp.float32),
        scratch_types=[
            pltpu.VMEM((3, _PW + 16), jnp.float32),
            pltpu.VMEM((4, _PW), jnp.float32),
            pltpu.VMEM((2 * _NL * 128,), jnp.int32),
            pltpu.VMEM((2 * _NL * 128,), jnp.float32),
            pltpu.VMEM((2 * _NL * 128, 8), jnp.float32),
            pltpu.VMEM((256, 32), jnp.float32),
            pltpu.SemaphoreType.DMA,
            pltpu.SemaphoreType.DMA,
        ],
        compiler_params=pltpu.CompilerParams(use_tc_tiling_on_sc=False,
                                            needs_layout_passes=False),
    )
    return lookup(Tt, xT, cT)
